# baseline (speedup 1.0000x reference)
import functools

import jax
import jax.numpy as jnp
import numpy as np
from jax.experimental import pallas as pl

_P = 16
_K = 10
_EPS = 1e-10
_CB = 256
_RB = 256
_PAD_COORD = 1e6
_IDX_MASK = 0xFFF
_INT_MAX = 0x7FFFFFFF
_SELF_W = float(np.float32(1.0) / (np.float32(0.0) + np.float32(_EPS)))


def _embed_body(xmid_ref, convw3_ref, convbc_ref, g_ref, nwrow_ref, nb_ref,
                featst_ref, gfeat_ref, *, cin):
    ft = None
    for c in range(cin):
        part = jax.lax.dot_general(convw3_ref[c], xmid_ref[0, c],
                                   (((0,), (1,)), ((), ())),
                                   preferred_element_type=jnp.float32)
        ft = part if ft is None else ft + part
    featst_ref[0] = (ft + convbc_ref[:, 0:1]).astype(jnp.bfloat16)
    g = g_ref[0][:, 0:1]
    gfeat_ref[0] = g * nwrow_ref[0:1, :] + nb_ref[0:1, :]


def _knn_body(lcol_ref, lrow_ref, wt_ref, *, k, rb, ng):
    yj = lcol_ref[:, 0:1]
    xj = lcol_ref[:, 1:2]
    yi = lrow_ref[0:1, :]
    xi = lrow_ref[1:2, :]
    dy = yj - yi
    dx = xj - xi
    d2 = dy * dy + dx * dx
    rows = jax.lax.broadcasted_iota(jnp.int32, d2.shape, 0)
    enc = (jax.lax.bitcast_convert_type(d2, jnp.int32) & ~_IDX_MASK) | rows
    qcols = (jax.lax.broadcasted_iota(jnp.int32, d2.shape, 1)
             + pl.program_id(0) * rb)
    enc = jnp.where(rows == qcols, _INT_MAX, enc)
    m = jnp.min(enc, axis=0, keepdims=True)
    for _ in range(k - 2):
        m = jnp.min(jnp.where(enc > m, enc, _INT_MAX), axis=0, keepdims=True)
    w = jax.lax.rsqrt(jnp.maximum(d2, 1e-20))
    acc = jnp.where(enc <= m, w, 0.0)
    wt_ref[...] = acc[:ng, :].astype(jnp.bfloat16)


def _gnn_body(wt_ref, xbf_ref, xself_ref, w1t_ref, b1_ref, w2t_ref, b2_ref,
              lns_ref, lnb_ref, ones_ref, out_ref, *, nt, f):
    wblk = wt_ref[...]
    ones = ones_ref[...]
    for t in range(nt):
        agg = jax.lax.dot_general(wblk, xbf_ref[t], (((1,), (0,)), ((), ())),
                                  preferred_element_type=jnp.float32)
        agg = agg + _SELF_W * xself_ref[t]
        h = jax.lax.dot_general(agg, w1t_ref[...], (((1,), (0,)), ((), ())),
                                preferred_element_type=jnp.float32)
        h = jnp.maximum(h + b1_ref[0:1, :], 0.0)
        o = jax.lax.dot_general(h, w2t_ref[...], (((1,), (0,)), ((), ())),
                                preferred_element_type=jnp.float32)
        o = o + b2_ref[0:1, :]
        s1 = jax.lax.dot_general(o, ones, (((1,), (0,)), ((), ())),
                                 preferred_element_type=jnp.float32)
        s2 = jax.lax.dot_general(o * o, ones, (((1,), (0,)), ((), ())),
                                 preferred_element_type=jnp.float32)
        mu = s1[:, 0:1] * (1.0 / f)
        var = s2[:, 0:1] * (1.0 / f) - mu * mu
        out_ref[t] = ((o - mu) * jax.lax.rsqrt(var + 1e-5) * lns_ref[0:1, :]
                      + lnb_ref[0:1, :])


def kernel(grid_data, graph_data, lat_lon_coords, graph_time_indices,
           grid_time_indices, conv_w, conv_b, node_w, node_b, gnn_w1,
           gnn_b1, gnn_w2, gnn_b2, ln_scale, ln_bias):
    B, T, CIN, H, W = grid_data.shape
    NG = graph_data.shape[2]
    F = conv_w.shape[0]
    HID = gnn_w1.shape[0]
    HP, WP = H // _P, W // _P
    NPAT = HP * WP
    CPP = CIN * _P * _P
    M = NG + NPAT
    MP = -(-M // _RB) * _RB

    y = jnp.linspace(0.0, 1.0, HP)
    x = jnp.linspace(0.0, 1.0, WP)
    yy, xx = jnp.meshgrid(y, x, indexing="ij")
    ploc = jnp.stack([yy, xx], axis=-1).reshape(-1, 2)

    PP = _P * _P
    convw3 = conv_w.T.astype(jnp.bfloat16).reshape(CIN, PP, F)
    convbc = jnp.pad(conv_b.reshape(F, 1), ((0, 0), (0, 7)))
    nwrow = node_w.reshape(1, F)
    nb2 = node_b.reshape(1, F)
    w1t = gnn_w1.T
    w2t = gnn_w2.T
    onescol = jnp.ones((F, 8), jnp.float32)
    b1 = gnn_b1.reshape(1, HID)
    b2 = gnn_b2.reshape(1, F)
    lns = ln_scale.reshape(1, F)
    lnb = ln_bias.reshape(1, F)

    embed_call = pl.pallas_call(
        functools.partial(_embed_body, cin=CIN),
        grid=(T,),
        in_specs=[
            pl.BlockSpec((1, CIN, NPAT, PP), lambda t: (t, 0, 0, 0)),
            pl.BlockSpec((CIN, PP, F), lambda t: (0, 0, 0)),
            pl.BlockSpec((F, 8), lambda t: (0, 0)),
            pl.BlockSpec((1, NG, 8), lambda t: (t, 0, 0)),
            pl.BlockSpec((1, F), lambda t: (0, 0)),
            pl.BlockSpec((1, F), lambda t: (0, 0)),
        ],
        out_specs=[
            pl.BlockSpec((1, F, NPAT), lambda t: (t, 0, 0)),
            pl.BlockSpec((1, NG, F), lambda t: (t, 0, 0)),
        ],
        out_shape=[
            jax.ShapeDtypeStruct((T, F, NPAT), jnp.bfloat16),
            jax.ShapeDtypeStruct((T, NG, F), jnp.float32),
        ],
    )

    knn_call = pl.pallas_call(
        functools.partial(_knn_body, k=_K, rb=_RB, ng=NG),
        grid=(MP // _RB,),
        in_specs=[
            pl.BlockSpec((MP, 8), lambda i: (0, 0)),
            pl.BlockSpec((8, _RB), lambda i: (0, i)),
        ],
        out_specs=pl.BlockSpec((NG, _RB), lambda i: (0, i)),
        out_shape=jax.ShapeDtypeStruct((NG, MP), jnp.bfloat16),
    )

    gnn_call = pl.pallas_call(
        functools.partial(_gnn_body, nt=T, f=float(F)),
        grid=(NG // _CB,),
        in_specs=[
            pl.BlockSpec((_CB, MP), lambda j: (j, 0)),
            pl.BlockSpec((T, MP, F), lambda j: (0, 0, 0)),
            pl.BlockSpec((T, _CB, F), lambda j: (0, j, 0)),
            pl.BlockSpec((F, HID), lambda j: (0, 0)),
            pl.BlockSpec((1, HID), lambda j: (0, 0)),
            pl.BlockSpec((HID, F), lambda j: (0, 0)),
            pl.BlockSpec((1, F), lambda j: (0, 0)),
            pl.BlockSpec((1, F), lambda j: (0, 0)),
            pl.BlockSpec((1, F), lambda j: (0, 0)),
            pl.BlockSpec((F, 8), lambda j: (0, 0)),
        ],
        out_specs=pl.BlockSpec((T, _CB, F), lambda j: (0, j, 0)),
        out_shape=jax.ShapeDtypeStruct((T, NG, F), jnp.float32),
    )

    outs_b = []
    for b in range(B):
        xmid = (grid_data[b].astype(jnp.bfloat16)
                .reshape(T, CIN, HP, _P, WP, _P)
                .transpose(0, 1, 2, 4, 3, 5)
                .reshape(T, CIN, NPAT, PP))
        g8 = jnp.pad(graph_data[b], ((0, 0), (0, 0), (0, 7)))

        featst, gfeat = embed_call(xmid, convw3, convbc, g8, nwrow, nb2)
        patches = featst.reshape(T, NPAT, F)
        xbf = jnp.pad(
            jnp.concatenate([gfeat.astype(jnp.bfloat16), patches], axis=1),
            ((0, 0), (0, MP - M), (0, 0)))

        gloc = jnp.stack([(lat_lon_coords[b, :, 0] + 90.0) / 180.0,
                          (lat_lon_coords[b, :, 1] + 180.0) / 360.0], axis=-1)
        loc = jnp.concatenate([gloc, ploc], axis=0)
        loc = jnp.pad(loc, ((0, MP - M), (0, 0)),
                      constant_values=_PAD_COORD)
        lcol = jnp.pad(loc, ((0, 0), (0, 6)))
        lrow = jnp.pad(loc.T, ((0, 6), (0, 0)))

        wtmat = knn_call(lcol, lrow)
        out = gnn_call(wtmat, xbf, gfeat, w1t, b1, w2t, b2, lns, lnb, onescol)
        outs_b.append(out)
    return jnp.stack(outs_b, axis=0)
